# baseline (speedup 1.0000x reference)
import functools

import jax
import jax.numpy as jnp
from jax import lax
from jax.experimental import pallas as pl
from jax.experimental.pallas import tpu as pltpu
from jax.experimental.pallas import tpu_sc as plsc

N = 1000000
M = 1000000
K = 32
B = 16384

NC = 2
NS = 16
L = 16
NW = NC * NS
BPW = B // NW
CHUNK = 128
NCHUNK = BPW // CHUNK
NGROUP = BPW // L


def _sc_body(w_ref, u_ref, W_ref, U_ref, b_ref, c_ref, out_ref,
             idx_w, idx_u, we_v, ue_v, bw_v, cu_v, out_v, sem):
    wid = lax.axis_index("s") * NC + lax.axis_index("c")
    base = wid * BPW

    pltpu.sync_copy(w_ref.at[pl.ds(wid * NCHUNK, NCHUNK)], idx_w)
    pltpu.sync_copy(u_ref.at[pl.ds(wid * NCHUNK, NCHUNK)], idx_u)

    copies = []
    for j in range(NCHUNK):
        copies.append(pltpu.async_copy(
            W_ref.at[idx_w.at[j]], we_v.at[pl.ds(j * CHUNK, CHUNK)], sem))
        copies.append(pltpu.async_copy(
            U_ref.at[idx_u.at[j]], ue_v.at[pl.ds(j * CHUNK, CHUNK)], sem))
        copies.append(pltpu.async_copy(
            b_ref.at[idx_w.at[j]], bw_v.at[pl.ds(j * CHUNK, CHUNK)], sem))
        copies.append(pltpu.async_copy(
            c_ref.at[idx_u.at[j]], cu_v.at[pl.ds(j * CHUNK, CHUNK)], sem))
    for cp in copies:
        cp.wait()

    lanes = lax.iota(jnp.int32, L)

    def g_body(g, carry):
        rows = lanes + g * L
        acc = bw_v[pl.ds(g * L, L)] + cu_v[pl.ds(g * L, L)]
        for k in range(K):
            kidx = jnp.full((L,), k, jnp.int32)
            acc = acc + (plsc.load_gather(we_v, [rows, kidx])
                         * plsc.load_gather(ue_v, [rows, kidx]))
        out_v[pl.ds(g * L, L)] = acc
        return carry

    lax.fori_loop(0, NGROUP, g_body, 0)
    pltpu.sync_copy(out_v, out_ref.at[pl.ds(base, BPW)])


@functools.partial(jax.jit, static_argnames=())
def kernel(w, u, W_emb, U_emb, b_emb, c_emb):
    mesh = plsc.VectorSubcoreMesh(
        core_axis_name="c", subcore_axis_name="s",
        num_cores=NC, num_subcores=NS)
    run = pl.kernel(
        _sc_body,
        out_type=jax.ShapeDtypeStruct((B,), jnp.float32),
        mesh=mesh,
        scratch_types=[
            pltpu.VMEM((NCHUNK, CHUNK), jnp.int32),
            pltpu.VMEM((NCHUNK, CHUNK), jnp.int32),
            pltpu.VMEM((BPW, K), jnp.float32),
            pltpu.VMEM((BPW, K), jnp.float32),
            pltpu.VMEM((BPW,), jnp.float32),
            pltpu.VMEM((BPW,), jnp.float32),
            pltpu.VMEM((BPW,), jnp.float32),
            pltpu.SemaphoreType.DMA,
        ],
        compiler_params=pltpu.CompilerParams(
            needs_layout_passes=False, use_tc_tiling_on_sc=False),
    )
    out = run(
        w.reshape(B // CHUNK, CHUNK),
        u.reshape(B // CHUNK, CHUNK),
        W_emb,
        U_emb,
        b_emb.reshape(N),
        c_emb.reshape(M),
    )
    return out.reshape(B, 1, 1)


# speedup vs baseline: 2.4970x; 2.4970x over previous
import jax
import jax.numpy as jnp
from jax import lax
from jax.experimental import pallas as pl
from jax.experimental.pallas import tpu as pltpu
from jax.experimental.pallas import tpu_sc as plsc

N = 1000000
M = 1000000
K = 32
B = 16384

NC = 2
NS = 16
L = 16

RMAX = 999936
NTAIL = 128
RW = 40960
NROUND = 25
CHT = RW // NS
EPT = B // NS
NG = EPT // L


def _k1_body(w_ref, u_ref, Wt_ref, Ut_ref, wt_tail_ref, ut_tail_ref,
             p0_o, p1_o,
             idx_w, idx_u, wwin, uwin,
             req_rw, req_iw, req_ru, req_iu, wrow, urow, out_v, sem):
    c = lax.axis_index("c")
    s = lax.axis_index("s")
    kbase = c * (K // 2)
    ebase = s * EPT

    pltpu.sync_copy(w_ref.at[pl.ds(s * 8, 8)], idx_w)
    pltpu.sync_copy(u_ref.at[pl.ds(s * 8, 8)], idx_u)

    @pl.when(s == 0)
    def _():
        for a in range(2):
            pltpu.sync_copy(wt_tail_ref.at[pl.ds(kbase + 8 * a, 8), pl.ds(0, NTAIL)],
                            wwin.at[pl.ds(8 * a, 8), pl.ds(RW, NTAIL)])
            pltpu.sync_copy(ut_tail_ref.at[pl.ds(kbase + 8 * a, 8), pl.ds(0, NTAIL)],
                            uwin.at[pl.ds(8 * a, 8), pl.ds(RW, NTAIL)])

    lanes = lax.iota(jnp.int32, L)

    def round_body(p, carry):
        lo = p * RW
        hi = jnp.minimum(lo + RW, N)
        off = jnp.minimum(lo + s * CHT, RMAX - CHT)
        off = pl.multiple_of(off, 128)
        cps = []
        for a in range(2):
            cps.append(pltpu.async_copy(
                Wt_ref.at[pl.ds(kbase + 8 * a, 8), pl.ds(off, CHT)],
                wwin.at[pl.ds(8 * a, 8), pl.ds(s * CHT, CHT)], sem))
            cps.append(pltpu.async_copy(
                Ut_ref.at[pl.ds(kbase + 8 * a, 8), pl.ds(off, CHT)],
                uwin.at[pl.ds(8 * a, 8), pl.ds(s * CHT, CHT)], sem))
        for cp in cps:
            cp.wait()
        plsc.subcore_barrier()

        def compact(idx_ref, rq_r, rq_i):
            def grp(g, off_c):
                row = g // 8
                col = (g % 8) * L
                v = idx_ref[row, pl.ds(col, L)]
                m = (v >= lo) & (v < hi)
                mi = m.astype(jnp.int32)
                pos = plsc.cumsum(mi) - 1 + off_c
                il = lanes + g * L
                plsc.store_scatter(rq_r, [pos], v, mask=m)
                plsc.store_scatter(rq_i, [pos], il, mask=m)
                return off_c + jax.lax.reduce_sum(mi, axes=(0,))
            cnt = lax.fori_loop(0, NG, grp, jnp.int32(0))
            rq_r[pl.ds(cnt, L)] = jnp.full((L,), lo, jnp.int32)
            rq_i[pl.ds(cnt, L)] = jnp.full((L,), EPT, jnp.int32)
            return cnt

        wcnt = compact(idx_w, req_rw, req_iw)
        ucnt = compact(idx_u, req_ru, req_iu)

        def serve(rq_r, rq_i, cnt, win, row):
            nch = (cnt + L - 1) // L

            def chunk(q, carry2):
                r16 = rq_r[pl.ds(q * L, L)]
                i16 = rq_i[pl.ds(q * L, L)]
                for j in range(L):
                    r = r16[j]
                    il = i16[j]
                    t = (r - lo) // CHT
                    src0 = jnp.minimum(lo + t * CHT, RMAX - CHT)
                    wcol = jnp.where(r >= RMAX,
                                     RW + (r - RMAX),
                                     t * CHT + (r - src0))
                    for a in range(2):
                        pltpu.async_copy(
                            win.at[pl.ds(8 * a, 8), pl.ds(wcol, 1)],
                            row.at[pl.ds(8 * a, 8), pl.ds(il, 1)], sem)
                return carry2

            lax.fori_loop(0, nch, chunk, 0)

            def dchunk(q, carry2):
                for _ in range(2 * L):
                    pltpu.make_async_copy(
                        win.at[pl.ds(0, 8), pl.ds(0, 1)],
                        row.at[pl.ds(0, 8), pl.ds(EPT, 1)], sem).wait()
                return carry2

            lax.fori_loop(0, nch, dchunk, 0)

        serve(req_rw, req_iw, wcnt, wwin, wrow)
        serve(req_ru, req_iu, ucnt, uwin, urow)
        plsc.subcore_barrier()
        return carry

    lax.fori_loop(0, NROUND, round_body, 0)

    def g_body(g, carry):
        il16 = lanes + g * L
        acc = jnp.zeros((L,), jnp.float32)
        for k in range(K // 2):
            ks = jnp.full((L,), k, jnp.int32)
            acc = acc + (plsc.load_gather(wrow, [ks, il16])
                         * plsc.load_gather(urow, [ks, il16]))
        out_v[pl.ds(g * L, L)] = acc
        return carry

    lax.fori_loop(0, NG, g_body, 0)

    @pl.when(c == 0)
    def _():
        pltpu.sync_copy(out_v, p0_o.at[pl.ds(ebase, EPT)])

    @pl.when(c == 1)
    def _():
        pltpu.sync_copy(out_v, p1_o.at[pl.ds(ebase, EPT)])


def _k2_body(w_ref, u_ref, p0_ref, p1_ref, b_ref, c_ref, out_ref,
             idx_w, idx_u, p0_v, p1_v, bw_v, cu_v, out_v, sem):
    NW = NC * NS
    BPW = B // NW
    wid = lax.axis_index("s") * NC + lax.axis_index("c")
    base = wid * BPW

    pltpu.sync_copy(w_ref.at[pl.ds(wid * 4, 4)], idx_w)
    pltpu.sync_copy(u_ref.at[pl.ds(wid * 4, 4)], idx_u)
    cps = [
        pltpu.async_copy(p0_ref.at[pl.ds(base, BPW)], p0_v, sem),
        pltpu.async_copy(p1_ref.at[pl.ds(base, BPW)], p1_v, sem),
    ]
    for j in range(4):
        cps.append(pltpu.async_copy(
            b_ref.at[idx_w.at[j]], bw_v.at[pl.ds(j * 128, 128)], sem))
        cps.append(pltpu.async_copy(
            c_ref.at[idx_u.at[j]], cu_v.at[pl.ds(j * 128, 128)], sem))
    for cp in cps:
        cp.wait()

    def g_body(g, carry):
        sl = pl.ds(g * L, L)
        out_v[sl] = p0_v[sl] + p1_v[sl] + bw_v[sl] + cu_v[sl]
        return carry

    lax.fori_loop(0, BPW // L, g_body, 0)
    pltpu.sync_copy(out_v, out_ref.at[pl.ds(base, BPW)])


def kernel(w, u, W_emb, U_emb, b_emb, c_emb):
    mesh = plsc.VectorSubcoreMesh(
        core_axis_name="c", subcore_axis_name="s",
        num_cores=NC, num_subcores=NS)

    k1 = pl.kernel(
        _k1_body,
        out_type=[jax.ShapeDtypeStruct((B,), jnp.float32)] * 2,
        mesh=mesh,
        scratch_types=[
            pltpu.VMEM((8, 128), jnp.int32),
            pltpu.VMEM((8, 128), jnp.int32),
            pltpu.VMEM_SHARED((16, RW + NTAIL), jnp.float32),
            pltpu.VMEM_SHARED((16, RW + NTAIL), jnp.float32),
            pltpu.VMEM((EPT + 2 * L,), jnp.int32),
            pltpu.VMEM((EPT + 2 * L,), jnp.int32),
            pltpu.VMEM((EPT + 2 * L,), jnp.int32),
            pltpu.VMEM((EPT + 2 * L,), jnp.int32),
            pltpu.VMEM((16, EPT + 2), jnp.float32),
            pltpu.VMEM((16, EPT + 2), jnp.float32),
            pltpu.VMEM((EPT,), jnp.float32),
            pltpu.SemaphoreType.DMA,
        ],
        compiler_params=pltpu.CompilerParams(
            needs_layout_passes=False, use_tc_tiling_on_sc=True),
    )
    w2d = w.reshape(B // 128, 128)
    u2d = u.reshape(B // 128, 128)
    wt_tail = jnp.zeros((K, NTAIL), jnp.float32).at[:, :N - RMAX].set(W_emb[RMAX:].T)
    ut_tail = jnp.zeros((K, NTAIL), jnp.float32).at[:, :M - RMAX].set(U_emb[RMAX:].T)
    p0, p1 = k1(w2d, u2d, W_emb.T, U_emb.T, wt_tail, ut_tail)

    k2 = pl.kernel(
        _k2_body,
        out_type=jax.ShapeDtypeStruct((B,), jnp.float32),
        mesh=mesh,
        scratch_types=[
            pltpu.VMEM((4, 128), jnp.int32),
            pltpu.VMEM((4, 128), jnp.int32),
            pltpu.VMEM((512,), jnp.float32),
            pltpu.VMEM((512,), jnp.float32),
            pltpu.VMEM((512,), jnp.float32),
            pltpu.VMEM((512,), jnp.float32),
            pltpu.VMEM((512,), jnp.float32),
            pltpu.SemaphoreType.DMA,
        ],
        compiler_params=pltpu.CompilerParams(
            needs_layout_passes=False, use_tc_tiling_on_sc=False),
    )
    out = k2(w2d, u2d, p0, p1, b_emb.reshape(N), c_emb.reshape(M))
    return out.reshape(B, 1, 1)


# speedup vs baseline: 2.5016x; 1.0018x over previous
import jax
import jax.numpy as jnp
from jax import lax
from jax.experimental import pallas as pl
from jax.experimental.pallas import tpu as pltpu
from jax.experimental.pallas import tpu_sc as plsc

N = 1000000
M = 1000000
K = 32
B = 16384

NC = 2
NS = 16
L = 16

RMAX = 999936
NTAIL = 128
RW = 32768
NROUND = 31
CHT = RW // NS
EPT = B // NS
NG = EPT // L


def _k1_body(w_ref, u_ref, Wt_ref, Ut_ref, wt_tail_ref, ut_tail_ref,
             p0_o, p1_o,
             idx_w, idx_u, wwin, uwin,
             req_rw, req_iw, req_ru, req_iu, wrow, urow, out_v, sem):
    c = lax.axis_index("c")
    s = lax.axis_index("s")
    kbase = c * (K // 2)
    ebase = s * EPT

    pltpu.sync_copy(w_ref.at[pl.ds(s * 8, 8)], idx_w)
    pltpu.sync_copy(u_ref.at[pl.ds(s * 8, 8)], idx_u)

    @pl.when(s == 0)
    def _():
        for a in range(2):
            pltpu.sync_copy(wt_tail_ref.at[pl.ds(kbase + 8 * a, 8), pl.ds(0, NTAIL)],
                            wwin.at[pl.ds(8 * a, 8), pl.ds(RW, NTAIL)])
            pltpu.sync_copy(ut_tail_ref.at[pl.ds(kbase + 8 * a, 8), pl.ds(0, NTAIL)],
                            uwin.at[pl.ds(8 * a, 8), pl.ds(RW, NTAIL)])

    lanes = lax.iota(jnp.int32, L)

    def round_body(p, carry):
        lo = p * RW
        hi = jnp.minimum(lo + RW, N)
        off = jnp.minimum(lo + s * CHT, RMAX - CHT)
        off = pl.multiple_of(off, 128)
        cps = []
        for a in range(2):
            cps.append(pltpu.async_copy(
                Wt_ref.at[pl.ds(kbase + 8 * a, 8), pl.ds(off, CHT)],
                wwin.at[pl.ds(8 * a, 8), pl.ds(s * CHT, CHT)], sem))
            cps.append(pltpu.async_copy(
                Ut_ref.at[pl.ds(kbase + 8 * a, 8), pl.ds(off, CHT)],
                uwin.at[pl.ds(8 * a, 8), pl.ds(s * CHT, CHT)], sem))
        for cp in cps:
            cp.wait()
        plsc.subcore_barrier()

        def compact(idx_ref, rq_r, rq_i):
            def grp(g, off_c):
                row = g // 8
                col = (g % 8) * L
                v = idx_ref[row, pl.ds(col, L)]
                m = (v >= lo) & (v < hi)
                mi = m.astype(jnp.int32)
                pos = plsc.cumsum(mi) - 1 + off_c
                il = lanes + g * L
                plsc.store_scatter(rq_r, [pos], v, mask=m)
                plsc.store_scatter(rq_i, [pos], il, mask=m)
                return off_c + jax.lax.reduce_sum(mi, axes=(0,))
            cnt = lax.fori_loop(0, NG, grp, jnp.int32(0))
            rq_r[pl.ds(cnt, L)] = jnp.full((L,), lo, jnp.int32)
            rq_i[pl.ds(cnt, L)] = jnp.full((L,), EPT, jnp.int32)
            return cnt

        wcnt = compact(idx_w, req_rw, req_iw)
        ucnt = compact(idx_u, req_ru, req_iu)

        def serve(rq_r, rq_i, cnt, win, row):
            nch = (cnt + L - 1) // L

            def chunk(q, carry2):
                r16 = rq_r[pl.ds(q * L, L)]
                i16 = rq_i[pl.ds(q * L, L)]
                t = (r16 - lo) // CHT
                src0 = jnp.minimum(lo + t * CHT, RMAX - CHT)
                wcol = jnp.where(r16 >= RMAX,
                                 RW + (r16 - RMAX),
                                 t * CHT + (r16 - src0))
                for j in range(L):
                    wc = wcol[j]
                    il = i16[j]
                    for a in range(2):
                        pltpu.async_copy(
                            win.at[pl.ds(8 * a, 8), pl.ds(wc, 1)],
                            row.at[pl.ds(8 * a, 8), pl.ds(il, 1)], sem)
                return carry2

            lax.fori_loop(0, nch, chunk, 0)

            def dchunk(q, carry2):
                for _ in range(2 * L):
                    pltpu.make_async_copy(
                        win.at[pl.ds(0, 8), pl.ds(0, 1)],
                        row.at[pl.ds(0, 8), pl.ds(EPT, 1)], sem).wait()
                return carry2

            lax.fori_loop(0, nch, dchunk, 0)

        serve(req_rw, req_iw, wcnt, wwin, wrow)
        serve(req_ru, req_iu, ucnt, uwin, urow)
        plsc.subcore_barrier()
        return carry

    lax.fori_loop(0, NROUND, round_body, 0)

    def g_body(g, carry):
        il16 = lanes + g * L
        acc = jnp.zeros((L,), jnp.float32)
        for k in range(K // 2):
            ks = jnp.full((L,), k, jnp.int32)
            acc = acc + (plsc.load_gather(wrow, [ks, il16])
                         * plsc.load_gather(urow, [ks, il16]))
        out_v[pl.ds(g * L, L)] = acc
        return carry

    lax.fori_loop(0, NG, g_body, 0)

    @pl.when(c == 0)
    def _():
        pltpu.sync_copy(out_v, p0_o.at[pl.ds(ebase, EPT)])

    @pl.when(c == 1)
    def _():
        pltpu.sync_copy(out_v, p1_o.at[pl.ds(ebase, EPT)])


def _k2_body(w_ref, u_ref, p0_ref, p1_ref, b_ref, c_ref, out_ref,
             idx_w, idx_u, p0_v, p1_v, bw_v, cu_v, out_v, sem):
    NW = NC * NS
    BPW = B // NW
    wid = lax.axis_index("s") * NC + lax.axis_index("c")
    base = wid * BPW

    pltpu.sync_copy(w_ref.at[pl.ds(wid * 4, 4)], idx_w)
    pltpu.sync_copy(u_ref.at[pl.ds(wid * 4, 4)], idx_u)
    cps = [
        pltpu.async_copy(p0_ref.at[pl.ds(base, BPW)], p0_v, sem),
        pltpu.async_copy(p1_ref.at[pl.ds(base, BPW)], p1_v, sem),
    ]
    for j in range(4):
        cps.append(pltpu.async_copy(
            b_ref.at[idx_w.at[j]], bw_v.at[pl.ds(j * 128, 128)], sem))
        cps.append(pltpu.async_copy(
            c_ref.at[idx_u.at[j]], cu_v.at[pl.ds(j * 128, 128)], sem))
    for cp in cps:
        cp.wait()

    def g_body(g, carry):
        sl = pl.ds(g * L, L)
        out_v[sl] = p0_v[sl] + p1_v[sl] + bw_v[sl] + cu_v[sl]
        return carry

    lax.fori_loop(0, BPW // L, g_body, 0)
    pltpu.sync_copy(out_v, out_ref.at[pl.ds(base, BPW)])


def kernel(w, u, W_emb, U_emb, b_emb, c_emb):
    mesh = plsc.VectorSubcoreMesh(
        core_axis_name="c", subcore_axis_name="s",
        num_cores=NC, num_subcores=NS)

    k1 = pl.kernel(
        _k1_body,
        out_type=[jax.ShapeDtypeStruct((B,), jnp.float32)] * 2,
        mesh=mesh,
        scratch_types=[
            pltpu.VMEM((8, 128), jnp.int32),
            pltpu.VMEM((8, 128), jnp.int32),
            pltpu.VMEM_SHARED((16, RW + NTAIL), jnp.float32),
            pltpu.VMEM_SHARED((16, RW + NTAIL), jnp.float32),
            pltpu.VMEM((EPT + 2 * L,), jnp.int32),
            pltpu.VMEM((EPT + 2 * L,), jnp.int32),
            pltpu.VMEM((EPT + 2 * L,), jnp.int32),
            pltpu.VMEM((EPT + 2 * L,), jnp.int32),
            pltpu.VMEM((16, EPT + 2), jnp.float32),
            pltpu.VMEM((16, EPT + 2), jnp.float32),
            pltpu.VMEM((EPT,), jnp.float32),
            pltpu.SemaphoreType.DMA,
        ],
        compiler_params=pltpu.CompilerParams(
            needs_layout_passes=False, use_tc_tiling_on_sc=True),
    )
    w2d = w.reshape(B // 128, 128)
    u2d = u.reshape(B // 128, 128)
    wt_tail = jnp.zeros((K, NTAIL), jnp.float32).at[:, :N - RMAX].set(W_emb[RMAX:].T)
    ut_tail = jnp.zeros((K, NTAIL), jnp.float32).at[:, :M - RMAX].set(U_emb[RMAX:].T)
    p0, p1 = k1(w2d, u2d, W_emb.T, U_emb.T, wt_tail, ut_tail)

    k2 = pl.kernel(
        _k2_body,
        out_type=jax.ShapeDtypeStruct((B,), jnp.float32),
        mesh=mesh,
        scratch_types=[
            pltpu.VMEM((4, 128), jnp.int32),
            pltpu.VMEM((4, 128), jnp.int32),
            pltpu.VMEM((512,), jnp.float32),
            pltpu.VMEM((512,), jnp.float32),
            pltpu.VMEM((512,), jnp.float32),
            pltpu.VMEM((512,), jnp.float32),
            pltpu.VMEM((512,), jnp.float32),
            pltpu.SemaphoreType.DMA,
        ],
        compiler_params=pltpu.CompilerParams(
            needs_layout_passes=False, use_tc_tiling_on_sc=False),
    )
    out = k2(w2d, u2d, p0, p1, b_emb.reshape(N), c_emb.reshape(M))
    return out.reshape(B, 1, 1)


# speedup vs baseline: 2.9488x; 1.1788x over previous
import jax
import jax.numpy as jnp
from jax import lax
from jax.experimental import pallas as pl
from jax.experimental.pallas import tpu as pltpu
from jax.experimental.pallas import tpu_sc as plsc

N = 1000000
M = 1000000
K = 32
B = 16384

NC = 2
NS = 16
L = 16

RMAX = 999936
NTAIL = 128
RW = 32768
NROUND = 31
CHT = RW // NS
EPT = B // NS
NG = EPT // L


def _k1_body(w_ref, u_ref, Wt_ref, Ut_ref, wt_tail_ref, ut_tail_ref,
             p0_o, p1_o,
             idx_w, idx_u, wwin, uwin,
             req_rw, req_iw, req_ru, req_iu, wrow, urow, out_v, sem, semu):
    c = lax.axis_index("c")
    s = lax.axis_index("s")
    kbase = c * (K // 2)
    ebase = s * EPT

    pltpu.sync_copy(w_ref.at[pl.ds(s * 8, 8)], idx_w)
    pltpu.sync_copy(u_ref.at[pl.ds(s * 8, 8)], idx_u)

    @pl.when(s == 0)
    def _():
        for a in range(2):
            pltpu.sync_copy(wt_tail_ref.at[pl.ds(kbase + 8 * a, 8), pl.ds(0, NTAIL)],
                            wwin.at[pl.ds(8 * a, 8), pl.ds(RW, NTAIL)])
            pltpu.sync_copy(ut_tail_ref.at[pl.ds(kbase + 8 * a, 8), pl.ds(0, NTAIL)],
                            uwin.at[pl.ds(8 * a, 8), pl.ds(RW, NTAIL)])

    lanes = lax.iota(jnp.int32, L)

    def round_body(p, carry):
        lo = p * RW
        hi = jnp.minimum(lo + RW, N)
        off = jnp.minimum(lo + s * CHT, RMAX - CHT)
        off = pl.multiple_of(off, 128)
        cpw = []
        cpu = []
        for a in range(2):
            cpw.append(pltpu.async_copy(
                Wt_ref.at[pl.ds(kbase + 8 * a, 8), pl.ds(off, CHT)],
                wwin.at[pl.ds(8 * a, 8), pl.ds(s * CHT, CHT)], sem))
            cpu.append(pltpu.async_copy(
                Ut_ref.at[pl.ds(kbase + 8 * a, 8), pl.ds(off, CHT)],
                uwin.at[pl.ds(8 * a, 8), pl.ds(s * CHT, CHT)], semu))

        def compact(idx_ref, rq_r, rq_i):
            def grp(g, off_c):
                row = g // 8
                col = (g % 8) * L
                v = idx_ref[row, pl.ds(col, L)]
                m = (v >= lo) & (v < hi)
                mi = m.astype(jnp.int32)
                pos = plsc.cumsum(mi) - 1 + off_c
                il = lanes + g * L
                plsc.store_scatter(rq_r, [pos], v, mask=m)
                plsc.store_scatter(rq_i, [pos], il, mask=m)
                return off_c + jax.lax.reduce_sum(mi, axes=(0,))
            cnt = lax.fori_loop(0, NG, grp, jnp.int32(0))
            rq_r[pl.ds(cnt, L)] = jnp.full((L,), lo, jnp.int32)
            rq_i[pl.ds(cnt, L)] = jnp.full((L,), EPT, jnp.int32)
            return cnt

        wcnt = compact(idx_w, req_rw, req_iw)
        ucnt = compact(idx_u, req_ru, req_iu)

        def serve(rq_r, rq_i, cnt, win, row, ssem):
            nch = (cnt + L - 1) // L

            def chunk(q, carry2):
                r16 = rq_r[pl.ds(q * L, L)]
                i16 = rq_i[pl.ds(q * L, L)]
                t = (r16 - lo) // CHT
                src0 = jnp.minimum(lo + t * CHT, RMAX - CHT)
                wcol = jnp.where(r16 >= RMAX,
                                 RW + (r16 - RMAX),
                                 t * CHT + (r16 - src0))
                for j in range(L):
                    wc = wcol[j]
                    il = i16[j]
                    for a in range(2):
                        pltpu.async_copy(
                            win.at[pl.ds(8 * a, 8), pl.ds(wc, 1)],
                            row.at[pl.ds(8 * a, 8), pl.ds(il, 1)], ssem)
                return carry2

            lax.fori_loop(0, nch, chunk, 0)

            def dchunk(q, carry2):
                for _ in range(2 * L):
                    pltpu.make_async_copy(
                        win.at[pl.ds(0, 8), pl.ds(0, 1)],
                        row.at[pl.ds(0, 8), pl.ds(EPT, 1)], ssem).wait()
                return carry2

            lax.fori_loop(0, nch, dchunk, 0)

        for cp in cpw:
            cp.wait()
        plsc.subcore_barrier()
        serve(req_rw, req_iw, wcnt, wwin, wrow, sem)
        for cp in cpu:
            cp.wait()
        plsc.subcore_barrier()
        serve(req_ru, req_iu, ucnt, uwin, urow, semu)
        plsc.subcore_barrier()
        return carry

    lax.fori_loop(0, NROUND, round_body, 0)

    def g_body(g, carry):
        il16 = lanes + g * L
        acc = jnp.zeros((L,), jnp.float32)
        for k in range(K // 2):
            ks = jnp.full((L,), k, jnp.int32)
            acc = acc + (plsc.load_gather(wrow, [ks, il16])
                         * plsc.load_gather(urow, [ks, il16]))
        out_v[pl.ds(g * L, L)] = acc
        return carry

    lax.fori_loop(0, NG, g_body, 0)

    @pl.when(c == 0)
    def _():
        pltpu.sync_copy(out_v, p0_o.at[pl.ds(ebase, EPT)])

    @pl.when(c == 1)
    def _():
        pltpu.sync_copy(out_v, p1_o.at[pl.ds(ebase, EPT)])


def _k2_body(w_ref, u_ref, p0_ref, p1_ref, b_ref, c_ref, out_ref,
             idx_w, idx_u, p0_v, p1_v, bw_v, cu_v, out_v, sem):
    NW = NC * NS
    BPW = B // NW
    wid = lax.axis_index("s") * NC + lax.axis_index("c")
    base = wid * BPW

    pltpu.sync_copy(w_ref.at[pl.ds(wid * 4, 4)], idx_w)
    pltpu.sync_copy(u_ref.at[pl.ds(wid * 4, 4)], idx_u)
    cps = [
        pltpu.async_copy(p0_ref.at[pl.ds(base, BPW)], p0_v, sem),
        pltpu.async_copy(p1_ref.at[pl.ds(base, BPW)], p1_v, sem),
    ]
    for j in range(4):
        cps.append(pltpu.async_copy(
            b_ref.at[idx_w.at[j]], bw_v.at[pl.ds(j * 128, 128)], sem))
        cps.append(pltpu.async_copy(
            c_ref.at[idx_u.at[j]], cu_v.at[pl.ds(j * 128, 128)], sem))
    for cp in cps:
        cp.wait()

    def g_body(g, carry):
        sl = pl.ds(g * L, L)
        out_v[sl] = p0_v[sl] + p1_v[sl] + bw_v[sl] + cu_v[sl]
        return carry

    lax.fori_loop(0, BPW // L, g_body, 0)
    pltpu.sync_copy(out_v, out_ref.at[pl.ds(base, BPW)])


def kernel(w, u, W_emb, U_emb, b_emb, c_emb):
    mesh = plsc.VectorSubcoreMesh(
        core_axis_name="c", subcore_axis_name="s",
        num_cores=NC, num_subcores=NS)

    k1 = pl.kernel(
        _k1_body,
        out_type=[jax.ShapeDtypeStruct((B,), jnp.float32)] * 2,
        mesh=mesh,
        scratch_types=[
            pltpu.VMEM((8, 128), jnp.int32),
            pltpu.VMEM((8, 128), jnp.int32),
            pltpu.VMEM_SHARED((16, RW + NTAIL), jnp.float32),
            pltpu.VMEM_SHARED((16, RW + NTAIL), jnp.float32),
            pltpu.VMEM((EPT + 2 * L,), jnp.int32),
            pltpu.VMEM((EPT + 2 * L,), jnp.int32),
            pltpu.VMEM((EPT + 2 * L,), jnp.int32),
            pltpu.VMEM((EPT + 2 * L,), jnp.int32),
            pltpu.VMEM((16, EPT + 2), jnp.float32),
            pltpu.VMEM((16, EPT + 2), jnp.float32),
            pltpu.VMEM((EPT,), jnp.float32),
            pltpu.SemaphoreType.DMA,
            pltpu.SemaphoreType.DMA,
        ],
        compiler_params=pltpu.CompilerParams(
            needs_layout_passes=False, use_tc_tiling_on_sc=True),
    )
    w2d = w.reshape(B // 128, 128)
    u2d = u.reshape(B // 128, 128)
    wt_tail = jnp.zeros((K, NTAIL), jnp.float32).at[:, :N - RMAX].set(W_emb[RMAX:].T)
    ut_tail = jnp.zeros((K, NTAIL), jnp.float32).at[:, :M - RMAX].set(U_emb[RMAX:].T)
    p0, p1 = k1(w2d, u2d, W_emb.T, U_emb.T, wt_tail, ut_tail)

    k2 = pl.kernel(
        _k2_body,
        out_type=jax.ShapeDtypeStruct((B,), jnp.float32),
        mesh=mesh,
        scratch_types=[
            pltpu.VMEM((4, 128), jnp.int32),
            pltpu.VMEM((4, 128), jnp.int32),
            pltpu.VMEM((512,), jnp.float32),
            pltpu.VMEM((512,), jnp.float32),
            pltpu.VMEM((512,), jnp.float32),
            pltpu.VMEM((512,), jnp.float32),
            pltpu.VMEM((512,), jnp.float32),
            pltpu.SemaphoreType.DMA,
        ],
        compiler_params=pltpu.CompilerParams(
            needs_layout_passes=False, use_tc_tiling_on_sc=False),
    )
    out = k2(w2d, u2d, p0, p1, b_emb.reshape(N), c_emb.reshape(M))
    return out.reshape(B, 1, 1)
